# baseline (speedup 1.0000x reference)
import functools

import jax
import jax.numpy as jnp
from jax import lax
from jax.experimental import pallas as pl
from jax.experimental.pallas import tpu as pltpu
from jax.experimental.pallas import tpu_sc as plsc

VOCAB = 1000000
DIM = 64
BATCH = 16384
HIST = 20

NC = 2
NS = 16
NW = NC * NS
B = BATCH * HIST
K = 128
ROWS_PER_W = B // NW
NCHUNK = ROWS_PER_W // K
NBUF = 4

_mesh = plsc.VectorSubcoreMesh(core_axis_name="c", subcore_axis_name="s")


@functools.partial(
    pl.kernel,
    mesh=_mesh,
    out_type=jax.ShapeDtypeStruct((B, DIM), jnp.float32),
    scratch_types=[
        pltpu.VMEM((NCHUNK, K), jnp.int32),
        pltpu.VMEM((NBUF, K, DIM), jnp.float32),
    ] + [pltpu.SemaphoreType.DMA] * (2 * NBUF),
    compiler_params=pltpu.CompilerParams(use_tc_tiling_on_sc=False),
)
def _embed(table, idx, out, idx_v, rows, *sems):
    gsem = sems[:NBUF]
    ssem = sems[NBUF:]
    wid = lax.axis_index("s") * NC + lax.axis_index("c")
    base = wid * ROWS_PER_W

    pltpu.sync_copy(idx.at[wid], idx_v)

    for b in range(NBUF):
        pltpu.async_copy(table.at[idx_v.at[b]], rows.at[b], gsem[b])

    @pl.loop(0, NCHUNK, step=NBUF)
    def _group(g):
        for b in range(NBUF):
            j = g + b
            pltpu.make_async_copy(table.at[pl.ds(0, K)], rows.at[b], gsem[b]).wait()
            pltpu.async_copy(rows.at[b], out.at[pl.ds(base + j * K, K)], ssem[b])
        for b in range(NBUF):
            jn = g + b + NBUF

            @pl.when(jn < NCHUNK)
            def _(b=b, jn=jn):
                pltpu.make_async_copy(rows.at[b], out.at[pl.ds(0, K)], ssem[b]).wait()
                pltpu.async_copy(table.at[idx_v.at[jn]], rows.at[b], gsem[b])

    for b in range(NBUF):
        pltpu.make_async_copy(rows.at[b], out.at[pl.ds(0, K)], ssem[b]).wait()


def kernel(x, weight):
    idx = x.astype(jnp.int32).reshape(NW, NCHUNK, K)
    out = _embed(weight, idx)
    return out.reshape(BATCH, HIST, DIM)
